# baseline (speedup 1.0000x reference)
import jax
import jax.numpy as jnp
from jax import lax
from jax.experimental import pallas as pl
from jax.experimental.pallas import tpu as pltpu
from jax.experimental.pallas import tpu_sc as plsc

_BATCH = 4
_NUM_AGENTS = 4
_D0 = 128
_D1 = 64
_DOUT = _D0 + _D1
_ROWS = _BATCH * _NUM_AGENTS
_LANES = 16


def _root_gather_scs(qs0_hbm, qs1_hbm, out_hbm, sem):
    def body(n, best):
        used = n == 0
        agent_idx = jnp.where(n == 0, _NUM_AGENTS - 1, 0)
        horizon = jnp.int32(0)
        obs0 = jnp.where(n == 0, 0, -1)
        obs1 = jnp.where(n == 0, 0, -1)
        m = (
            used
            & (agent_idx == _NUM_AGENTS - 1)
            & (horizon == 0)
            & jnp.logical_not((obs0 == -1) & (obs1 == -1))
        )
        return jnp.where(m & (n < best), n, best)

    ridx = lax.fori_loop(0, _LANES, body, jnp.int32(2**30))

    copies = []
    for b in range(_BATCH):
        for a in range(_NUM_AGENTS):
            copies.append(
                pltpu.async_copy(
                    qs0_hbm.at[b, a, ridx, pl.ds(0, _D0)],
                    out_hbm.at[b, a, 0, pl.ds(0, _D0)],
                    sem,
                )
            )
            copies.append(
                pltpu.async_copy(
                    qs1_hbm.at[b, a, ridx, pl.ds(0, _D1)],
                    out_hbm.at[b, a, 0, pl.ds(_D0, _D1)],
                    sem,
                )
            )
    for c in copies:
        c.wait()


@jax.jit
def kernel(qs_0, qs_1):
    mesh = plsc.ScalarSubcoreMesh(axis_name="c", num_cores=1)
    out = pl.kernel(
        _root_gather_scs,
        mesh=mesh,
        out_type=jax.ShapeDtypeStruct((_BATCH, _NUM_AGENTS, 1, _DOUT), jnp.float32),
        scratch_types=[pltpu.SemaphoreType.DMA],
        compiler_params=pltpu.CompilerParams(
            needs_layout_passes=False,
            use_tc_tiling_on_sc=False,
        ),
    )(qs_0, qs_1)
    return out


# speedup vs baseline: 1.0308x; 1.0308x over previous
import jax
import jax.numpy as jnp
from jax import lax
from jax.experimental import pallas as pl
from jax.experimental.pallas import tpu as pltpu
from jax.experimental.pallas import tpu_sc as plsc

_BATCH = 4
_NUM_AGENTS = 4
_D0 = 128
_D1 = 64
_DOUT = _D0 + _D1
_ROWS = _BATCH * _NUM_AGENTS
_LANES = 16


def _root_gather_scs(qs0_hbm, qs1_hbm, out_hbm, sem):
    def body(n, best):
        used = n == 0
        agent_idx = jnp.where(n == 0, _NUM_AGENTS - 1, 0)
        horizon = jnp.int32(0)
        obs0 = jnp.where(n == 0, 0, -1)
        obs1 = jnp.where(n == 0, 0, -1)
        m = (
            used
            & (agent_idx == _NUM_AGENTS - 1)
            & (horizon == 0)
            & jnp.logical_not((obs0 == -1) & (obs1 == -1))
        )
        return jnp.where(m & (n < best), n, best)

    ridx = lax.fori_loop(0, _LANES, body, jnp.int32(2**30))

    cp0 = pltpu.async_copy(
        qs0_hbm.at[ridx], out_hbm.at[:, pl.ds(0, _D0)], sem
    )
    cp1 = pltpu.async_copy(
        qs1_hbm.at[ridx], out_hbm.at[:, pl.ds(_D0, _D1)], sem
    )
    cp0.wait()
    cp1.wait()


@jax.jit
def kernel(qs_0, qs_1):
    qs0_nodes = qs_0.reshape(1, _ROWS, _D0)
    qs1_nodes = qs_1.reshape(1, _ROWS, _D1)
    mesh = plsc.ScalarSubcoreMesh(axis_name="c", num_cores=1)
    out = pl.kernel(
        _root_gather_scs,
        mesh=mesh,
        out_type=jax.ShapeDtypeStruct((_ROWS, _DOUT), jnp.float32),
        scratch_types=[pltpu.SemaphoreType.DMA],
        compiler_params=pltpu.CompilerParams(
            needs_layout_passes=False,
            use_tc_tiling_on_sc=False,
        ),
    )(qs0_nodes, qs1_nodes)
    return out.reshape(_BATCH, _NUM_AGENTS, 1, _DOUT)


# speedup vs baseline: 1.0380x; 1.0069x over previous
import jax
import jax.numpy as jnp
from jax import lax
from jax.experimental import pallas as pl
from jax.experimental.pallas import tpu as pltpu
from jax.experimental.pallas import tpu_sc as plsc

_BATCH = 4
_NUM_AGENTS = 4
_D0 = 128
_D1 = 64
_DOUT = _D0 + _D1
_ROWS = _BATCH * _NUM_AGENTS
_NODE_BLOCK = 16


def _root_gather_scs(qs0_hbm, qs1_hbm, out_hbm, sem):
    def body(n, best):
        used = n == 0
        agent_idx = jnp.where(n == 0, _NUM_AGENTS - 1, 0)
        horizon = jnp.int32(0)
        obs0 = jnp.where(n == 0, 0, -1)
        obs1 = jnp.where(n == 0, 0, -1)
        m = (
            used
            & (agent_idx == _NUM_AGENTS - 1)
            & (horizon == 0)
            & jnp.logical_not((obs0 == -1) & (obs1 == -1))
        )
        return jnp.where(m & (n < best), n, best)

    ridx = lax.fori_loop(0, _NODE_BLOCK, body, jnp.int32(2**30))

    cp0 = pltpu.async_copy(qs0_hbm.at[ridx], out_hbm.at[:, pl.ds(0, _D0)], sem)
    cp1 = pltpu.async_copy(qs1_hbm.at[ridx], out_hbm.at[:, pl.ds(_D0, _D1)], sem)
    cp0.wait()
    cp1.wait()


@jax.jit
def kernel(qs_0, qs_1):
    qs0_nodes = qs_0.reshape(1, _ROWS, _D0)
    qs1_nodes = qs_1.reshape(1, _ROWS, _D1)
    mesh = plsc.ScalarSubcoreMesh(axis_name="c", num_cores=1)
    out = pl.kernel(
        _root_gather_scs,
        mesh=mesh,
        out_type=jax.ShapeDtypeStruct((_ROWS, _DOUT), jnp.float32),
        scratch_types=[pltpu.SemaphoreType.DMA],
        compiler_params=pltpu.CompilerParams(
            needs_layout_passes=False,
            use_tc_tiling_on_sc=False,
        ),
    )(qs0_nodes, qs1_nodes)
    return out.reshape(_BATCH, _NUM_AGENTS, 1, _DOUT)
